# baseline (speedup 1.0000x reference)
import functools

import jax
import jax.numpy as jnp
from jax import lax
from jax.experimental import pallas as pl
from jax.experimental.pallas import tpu as pltpu
from jax.experimental.pallas import tpu_sc as plsc

B = 16384
L = 20
VOCAB = 30522
WORD_DIM = 300
HIDDEN = 128

PROJ_BLK = 512
VPAD = 30528
DUMMY = VOCAB
PROJ_GRID = (VPAD + PROJ_BLK - 1) // PROJ_BLK

PREP_BLK = 2048

NC = 2
NS = 16
LANES = 16
NW = NC * NS
RPW = B // NW
CH = 4
NCH = RPW // CH
NBUF = 2
GRP = 128
CHG = GRP // CH


def _proj_body(emb_ref, w_ref, b_ref, out_ref):
    i = pl.program_id(0)
    rows = i * PROJ_BLK + lax.broadcasted_iota(jnp.int32, (PROJ_BLK, 1), 0)
    valid = rows < VOCAB
    x = jnp.where(valid, emb_ref[...], 0.0)
    y = jnp.dot(x, w_ref[...], preferred_element_type=jnp.float32)
    y = jnp.maximum(y + b_ref[...], 0.0)
    out_ref[...] = jnp.where(valid, y, 0.0)


_project = pl.pallas_call(
    _proj_body,
    grid=(PROJ_GRID,),
    in_specs=[
        pl.BlockSpec((PROJ_BLK, WORD_DIM), lambda i: (i, 0)),
        pl.BlockSpec((WORD_DIM, HIDDEN), lambda i: (0, 0)),
        pl.BlockSpec((1, HIDDEN), lambda i: (0, 0)),
    ],
    out_specs=pl.BlockSpec((PROJ_BLK, HIDDEN), lambda i: (i, 0)),
    out_shape=jax.ShapeDtypeStruct((VPAD, HIDDEN), jnp.float32),
)


def _prep_body(tok_ref, remap_ref, inv_ref):
    t = tok_ref[...]
    m = t > 2
    remap_ref[...] = jnp.where(m, t, DUMMY)
    cnt = jnp.sum(m.astype(jnp.float32), axis=1, keepdims=True)
    inv_ref[...] = jnp.broadcast_to(1.0 / (cnt + 1e-10), (PREP_BLK, LANES))


_prep = pl.pallas_call(
    _prep_body,
    grid=(B // PREP_BLK,),
    in_specs=[pl.BlockSpec((PREP_BLK, L), lambda i: (i, 0))],
    out_specs=[
        pl.BlockSpec((PREP_BLK, L), lambda i: (i, 0)),
        pl.BlockSpec((PREP_BLK, LANES), lambda i: (i, 0)),
    ],
    out_shape=[
        jax.ShapeDtypeStruct((B, L), jnp.int32),
        jax.ShapeDtypeStruct((B, LANES), jnp.float32),
    ],
)


@functools.partial(
    pl.kernel,
    out_type=jax.ShapeDtypeStruct((B, HIDDEN), jnp.float32),
    mesh=plsc.VectorSubcoreMesh(core_axis_name="c", subcore_axis_name="s"),
    scratch_types=[
        pltpu.VMEM((RPW * L,), jnp.int32),
        pltpu.VMEM((RPW, LANES), jnp.float32),
        pltpu.VMEM((CH * L, HIDDEN), jnp.float32),
        pltpu.VMEM((CH * L, HIDDEN), jnp.float32),
        pltpu.VMEM((GRP, HIDDEN), jnp.float32),
        pltpu.SemaphoreType.DMA,
        pltpu.SemaphoreType.DMA,
    ],
)
def _sc_pool(table_hbm, tok_hbm, inv_hbm, out_hbm, tok_v, inv_v, buf0, buf1,
             out_v, sem0, sem1):
    wid = lax.axis_index("s") * NC + lax.axis_index("c")
    base = wid * RPW
    pltpu.sync_copy(tok_hbm.at[pl.ds(base * L, RPW * L)], tok_v)
    pltpu.sync_copy(inv_hbm.at[pl.ds(base, RPW)], inv_v)

    bufs = (buf0, buf1)
    sems = (sem0, sem1)

    def issue(c, buf, sem):
        off = pl.multiple_of(c * (CH * L), 8)
        pltpu.async_copy(table_hbm.at[tok_v.at[pl.ds(off, CH * L)]], buf, sem)

    for k in range(NBUF - 1):
        issue(k, bufs[k], sems[k])

    def pair(p, carry):
        for b in range(NBUF):
            c = p * NBUF + b
            buf, sem = bufs[b], sems[b]
            pltpu.make_async_copy(
                table_hbm.at[tok_v.at[pl.ds(0, CH * L)]], buf, sem
            ).wait()
            nxt = c + NBUF - 1
            nb = (b + NBUF - 1) % NBUF

            @pl.when(nxt < NCH)
            def _():
                issue(nxt, bufs[nb], sems[nb])

            cg = lax.rem(c, CHG)
            for r in range(CH):
                row = c * CH + r
                inv_vec = inv_v[row, :]
                for h in range(HIDDEN // LANES):
                    sl = pl.ds(h * LANES, LANES)
                    acc = buf[r * L, sl]
                    for j in range(1, L):
                        acc = acc + buf[r * L + j, sl]
                    out_v[cg * CH + r, sl] = acc * inv_vec

            @pl.when(cg == CHG - 1)
            def _():
                g = lax.div(c, CHG)
                pltpu.sync_copy(
                    out_v, out_hbm.at[pl.ds(base + g * GRP, GRP)]
                )
        return carry

    lax.fori_loop(0, NCH // NBUF, pair, 0)


def kernel(tokens, word_embeddings, W_fc, b_fc):
    table = _project(word_embeddings, W_fc, b_fc.reshape(1, HIDDEN))
    remap, inv = _prep(tokens)
    return _sc_pool(table, remap.reshape(B * L), inv)


# speedup vs baseline: 1.3540x; 1.3540x over previous
import functools

import jax
import jax.numpy as jnp
from jax import lax
from jax.experimental import pallas as pl
from jax.experimental.pallas import tpu as pltpu
from jax.experimental.pallas import tpu_sc as plsc

B = 16384
L = 20
VOCAB = 30522
WORD_DIM = 300
HIDDEN = 128

PROJ_BLK = 512
VPAD = 30528
DUMMY = VOCAB
PROJ_GRID = (VPAD + PROJ_BLK - 1) // PROJ_BLK

PREP_BLK = 2048

NC = 2
NS = 16
LANES = 16
NW = NC * NS
RPW = B // NW
CH = 4
NCH = RPW // CH
NBUF = 2
GRP = 128
CHG = GRP // CH


def _proj_body(emb_ref, w_ref, b_ref, out_ref):
    i = pl.program_id(0)
    rows = i * PROJ_BLK + lax.broadcasted_iota(jnp.int32, (PROJ_BLK, 1), 0)
    valid = rows < VOCAB
    x = jnp.where(valid, emb_ref[...], 0.0)
    y = jnp.dot(x, w_ref[...], preferred_element_type=jnp.float32)
    y = jnp.maximum(y + b_ref[...], 0.0)
    out_ref[...] = jnp.where(valid, y, 0.0)


_project = pl.pallas_call(
    _proj_body,
    grid=(PROJ_GRID,),
    in_specs=[
        pl.BlockSpec((PROJ_BLK, WORD_DIM), lambda i: (i, 0)),
        pl.BlockSpec((WORD_DIM, HIDDEN), lambda i: (0, 0)),
        pl.BlockSpec((1, HIDDEN), lambda i: (0, 0)),
    ],
    out_specs=pl.BlockSpec((PROJ_BLK, HIDDEN), lambda i: (i, 0)),
    out_shape=jax.ShapeDtypeStruct((VPAD, HIDDEN), jnp.float32),
)


def _prep_body(tok_ref, remap_ref, inv_ref):
    t = tok_ref[...]
    m = t > 2
    remap_ref[...] = jnp.where(m, t, DUMMY)
    cnt = jnp.sum(m.astype(jnp.float32), axis=1, keepdims=True)
    inv_ref[...] = jnp.broadcast_to(1.0 / (cnt + 1e-10), (PREP_BLK, LANES))


_prep = pl.pallas_call(
    _prep_body,
    grid=(B // PREP_BLK,),
    in_specs=[pl.BlockSpec((PREP_BLK, L), lambda i: (i, 0))],
    out_specs=[
        pl.BlockSpec((PREP_BLK, L), lambda i: (i, 0)),
        pl.BlockSpec((PREP_BLK, LANES), lambda i: (i, 0)),
    ],
    out_shape=[
        jax.ShapeDtypeStruct((B, L), jnp.int32),
        jax.ShapeDtypeStruct((B, LANES), jnp.float32),
    ],
)


@functools.partial(
    pl.kernel,
    out_type=jax.ShapeDtypeStruct((B, HIDDEN), jnp.float32),
    mesh=plsc.VectorSubcoreMesh(core_axis_name="c", subcore_axis_name="s"),
    scratch_types=[
        pltpu.VMEM((RPW * L,), jnp.int32),
        pltpu.VMEM((RPW, LANES), jnp.float32),
        pltpu.VMEM((CH * L, HIDDEN), jnp.float32),
        pltpu.VMEM((CH * L, HIDDEN), jnp.float32),
        pltpu.VMEM((GRP, HIDDEN), jnp.float32),
        pltpu.SemaphoreType.DMA,
        pltpu.SemaphoreType.DMA,
    ],
)
def _sc_pool(table_hbm, tok_hbm, inv_hbm, out_hbm, tok_v, inv_v, buf0, buf1,
             out_v, sem0, sem1):
    wid = lax.axis_index("s") * NC + lax.axis_index("c")
    base = wid * RPW
    pltpu.sync_copy(tok_hbm.at[pl.ds(base * L, RPW * L)], tok_v)
    pltpu.sync_copy(inv_hbm.at[pl.ds(base, RPW)], inv_v)

    bufs = (buf0, buf1)
    sems = (sem0, sem1)

    def issue(c, buf, sem):
        off = pl.multiple_of(c * (CH * L), 8)
        pltpu.async_copy(table_hbm.at[tok_v.at[pl.ds(off, CH * L)]], buf, sem)

    for k in range(NBUF - 1):
        issue(k, bufs[k], sems[k])

    def pair(p, carry):
        for b in range(NBUF):
            c = p * NBUF + b
            buf, sem = bufs[b], sems[b]
            pltpu.make_async_copy(
                table_hbm.at[tok_v.at[pl.ds(0, CH * L)]], buf, sem
            ).wait()
            nxt = c + NBUF - 1
            nb = (b + NBUF - 1) % NBUF

            @pl.when(nxt < NCH)
            def _():
                issue(nxt, bufs[nb], sems[nb])

            cg = lax.rem(c, CHG)
            NH = HIDDEN // LANES
            for r in range(CH):
                row = c * CH + r
                inv_vec = inv_v[row, :]
                accs = [buf[r * L, pl.ds(h * LANES, LANES)] for h in range(NH)]
                for j in range(1, L):
                    for h in range(NH):
                        accs[h] = accs[h] + buf[r * L + j,
                                                pl.ds(h * LANES, LANES)]
                for h in range(NH):
                    out_v[cg * CH + r, pl.ds(h * LANES, LANES)] = (
                        accs[h] * inv_vec)

            @pl.when(cg == CHG - 1)
            def _():
                g = lax.div(c, CHG)
                pltpu.sync_copy(
                    out_v, out_hbm.at[pl.ds(base + g * GRP, GRP)]
                )
        return carry

    lax.fori_loop(0, NCH // NBUF, pair, 0)


def kernel(tokens, word_embeddings, W_fc, b_fc):
    table = _project(word_embeddings, W_fc, b_fc.reshape(1, HIDDEN))
    remap, inv = _prep(tokens)
    return _sc_pool(table, remap.reshape(B * L), inv)


# speedup vs baseline: 1.5261x; 1.1271x over previous
import functools

import jax
import jax.numpy as jnp
from jax import lax
from jax.experimental import pallas as pl
from jax.experimental.pallas import tpu as pltpu
from jax.experimental.pallas import tpu_sc as plsc

B = 16384
L = 20
VOCAB = 30522
WORD_DIM = 300
HIDDEN = 128

PROJ_BLK = 2048
VPAD = 30528
PROJ_GRID = (VPAD + PROJ_BLK - 1) // PROJ_BLK

NC = 2
NS = 16
LANES = 16
NW = NC * NS
RPW = B // NW
CH = 4
NCH = RPW // CH
NBUF = 2
GRP = 128
CHG = GRP // CH
NH = HIDDEN // LANES


def _proj_body(emb_ref, w_ref, b_ref, out_ref):
    i = pl.program_id(0)
    rows = i * PROJ_BLK + lax.broadcasted_iota(jnp.int32, (PROJ_BLK, 1), 0)
    valid = jnp.logical_and(rows > 2, rows < VOCAB)
    x = jnp.where(valid, emb_ref[...], 0.0)
    y = jnp.dot(x, w_ref[...], preferred_element_type=jnp.float32)
    y = jnp.maximum(y + b_ref[...], 0.0)
    out_ref[...] = jnp.where(valid, y, 0.0)


_project = pl.pallas_call(
    _proj_body,
    grid=(PROJ_GRID,),
    in_specs=[
        pl.BlockSpec((PROJ_BLK, WORD_DIM), lambda i: (i, 0)),
        pl.BlockSpec((WORD_DIM, HIDDEN), lambda i: (0, 0)),
        pl.BlockSpec((1, HIDDEN), lambda i: (0, 0)),
    ],
    out_specs=pl.BlockSpec((PROJ_BLK, HIDDEN), lambda i: (i, 0)),
    out_shape=jax.ShapeDtypeStruct((VPAD, HIDDEN), jnp.float32),
)


@functools.partial(
    pl.kernel,
    out_type=jax.ShapeDtypeStruct((B, HIDDEN), jnp.float32),
    mesh=plsc.VectorSubcoreMesh(core_axis_name="c", subcore_axis_name="s"),
    scratch_types=[
        pltpu.VMEM((RPW * L,), jnp.int32),
        pltpu.VMEM((CH * L, HIDDEN), jnp.float32),
        pltpu.VMEM((CH * L, HIDDEN), jnp.float32),
        pltpu.VMEM((GRP, HIDDEN), jnp.float32),
        pltpu.SemaphoreType.DMA,
        pltpu.SemaphoreType.DMA,
    ],
)
def _sc_pool(table_hbm, tok_hbm, out_hbm, tok_v, bm0, bm1, out_v, sem0,
             sem1):
    wid = lax.axis_index("s") * NC + lax.axis_index("c")
    base = wid * RPW
    pltpu.sync_copy(tok_hbm.at[pl.ds(base * L, RPW * L)], tok_v)

    bufs = ((bm0, sem0), (bm1, sem1))

    def issue(c, bm, sem):
        off = pl.multiple_of(c * (CH * L), 8)
        pltpu.async_copy(table_hbm.at[tok_v.at[pl.ds(off, CH * L)]], bm, sem)

    for k in range(NBUF - 1):
        issue(k, *bufs[k])

    def pair(p, carry):
        for b in range(NBUF):
            c = p * NBUF + b
            bm, sem = bufs[b]
            pltpu.make_async_copy(
                table_hbm.at[tok_v.at[pl.ds(0, CH * L)]], bm, sem
            ).wait()
            nxt = c + NBUF - 1
            nb = (b + NBUF - 1) % NBUF

            @pl.when(nxt < NCH)
            def _():
                issue(nxt, *bufs[nb])

            off_t = pl.multiple_of(c * (CH * L), 8)
            ms = [
                jnp.where(tok_v[pl.ds(off_t + 16 * k, 16)] > 2, 1.0, 0.0)
                for k in range(5)
            ]
            cnts = []
            for r in range(CH):
                lanes = [ms[(r * L + j) // 16][(r * L + j) % 16]
                         for j in range(L)]
                cnt = lanes[0]
                for x in lanes[1:]:
                    cnt = cnt + x
                cnts.append(cnt)

            cg = lax.rem(c, CHG)
            for r in range(CH):
                recip = 1.0 / (
                    jnp.full((LANES,), cnts[r], jnp.float32) + 1e-10)
                accs = [bm[r * L, pl.ds(h * LANES, LANES)] for h in range(NH)]
                for j in range(1, L):
                    for h in range(NH):
                        accs[h] = accs[h] + bm[r * L + j,
                                               pl.ds(h * LANES, LANES)]
                for h in range(NH):
                    out_v[cg * CH + r, pl.ds(h * LANES, LANES)] = (
                        accs[h] * recip)

            @pl.when(cg == CHG - 1)
            def _():
                g = lax.div(c, CHG)
                pltpu.sync_copy(
                    out_v, out_hbm.at[pl.ds(base + g * GRP, GRP)]
                )
        return carry

    lax.fori_loop(0, NCH // NBUF, pair, 0)


def kernel(tokens, word_embeddings, W_fc, b_fc):
    table = _project(word_embeddings, W_fc, b_fc.reshape(1, HIDDEN))
    return _sc_pool(table, tokens.reshape(B * L))
